# baseline (speedup 1.0000x reference)
import functools

import jax
import jax.numpy as jnp
from jax import lax
from jax.experimental import pallas as pl
from jax.experimental.pallas import tpu as pltpu, tpu_sc as plsc

N = 4096
BH = BW = 32
GRID = N // BH
NNZ = 1638
NW = 32
PER_W = 52
RB = 13
NROUNDS = PER_W // RB
WIN = 4 * BW

_mesh = plsc.VectorSubcoreMesh(core_axis_name="c", subcore_axis_name="s")


@functools.partial(
    pl.kernel,
    out_type=jax.ShapeDtypeStruct((NNZ * BH, WIN), jnp.float32),
    mesh=_mesh,
    scratch_types=[
        pltpu.VMEM((128,), jnp.int32),
        pltpu.VMEM((128,), jnp.int32),
        pltpu.VMEM((2, RB * BH, WIN), jnp.float32),
        pltpu.SemaphoreType.DMA,
        pltpu.SemaphoreType.DMA,
        pltpu.SemaphoreType.DMA,
    ],
)
def _gather_windows(x, r2d, c2d, out, rows_v, cols_v, wide,
                    sem_in0, sem_in1, sem_out):
    wid = lax.axis_index("s") * 2 + lax.axis_index("c")
    pltpu.sync_copy(r2d.at[wid], rows_v)
    pltpu.sync_copy(c2d.at[wid], cols_v)

    rc = []
    for j in range((PER_W + 15) // 16):
        r16 = rows_v[pl.ds(j * 16, 16)]
        c16 = cols_v[pl.ds(j * 16, 16)]
        for k in range(16):
            if j * 16 + k >= PER_W:
                break
            rc.append((r16[k], c16[k]))

    def in_copy(g, m):
        r, c = rc[g * RB + m]
        return pltpu.make_async_copy(
            x.at[pl.ds(r * BH, BH), pl.ds((c >> 2) * WIN, WIN)],
            wide.at[g % 2, pl.ds(m * BH, BH)], sem_in1 if g % 2 else sem_in0)

    def out_copy(g):
        row0 = (wid * PER_W + g * RB) * BH
        return pltpu.make_async_copy(
            wide.at[g % 2], out.at[pl.ds(row0, RB * BH)], sem_out)

    for m in range(RB):
        in_copy(0, m).start()
    for g in range(NROUNDS):
        if g + 1 < NROUNDS:
            for m in range(RB):
                in_copy(g + 1, m).start()
        for m in range(RB):
            in_copy(g, m).wait()
        if g < 2:
            out_copy(g).start()
            out_copy(g).wait()
        else:
            @pl.when(wid < NW - 1)
            def _():
                out_copy(g).start()
                out_copy(g).wait()


_TC_GRID = 39
_TB = NNZ // _TC_GRID


def _extract_body(o_ref, in_ref, out_ref):
    x3 = in_ref[...].reshape(_TB, BH, WIN)
    i0 = pl.program_id(0) * _TB
    for nn in range(_TB):
        o = o_ref[i0 + nn]
        y = x3[nn, :, 0:BW]
        for q in (1, 2, 3):
            y = jnp.where(o == q, x3[nn, :, q * BW:(q + 1) * BW], y)
        out_ref[nn] = y


_extract_call = pl.pallas_call(
    _extract_body,
    grid=(_TC_GRID,),
    in_specs=[
        pl.BlockSpec(memory_space=pltpu.SMEM),
        pl.BlockSpec((_TB * BH, WIN), lambda i: (i, 0)),
    ],
    out_specs=pl.BlockSpec((_TB, BH, BW), lambda i: (i, 0, 0)),
    out_shape=jax.ShapeDtypeStruct((NNZ, BH, BW), jnp.float32),
)


def kernel(x, active_indices):
    ai = active_indices.astype(jnp.int32)
    pad = jnp.zeros((NW * PER_W, 2), jnp.int32).at[:NNZ].set(ai)
    r2d = jnp.zeros((NW, 128), jnp.int32).at[:, :PER_W].set(
        pad[:, 0].reshape(NW, PER_W))
    c2d = jnp.zeros((NW, 128), jnp.int32).at[:, :PER_W].set(
        pad[:, 1].reshape(NW, PER_W))
    wide = _gather_windows(x, r2d, c2d)
    return _extract_call(ai[:, 1] % 4, wide)


# speedup vs baseline: 1.9679x; 1.9679x over previous
import functools

import jax
import jax.numpy as jnp
from jax import lax
from jax.experimental import pallas as pl
from jax.experimental.pallas import tpu as pltpu, tpu_sc as plsc

N = 4096
BH = BW = 32
GRID = N // BH
NNZ = 1638
NW = 32
PER_W = 52
RB = 13
NROUNDS = PER_W // RB
WIN = 4 * BW

_mesh = plsc.VectorSubcoreMesh(core_axis_name="c", subcore_axis_name="s")


@functools.partial(
    pl.kernel,
    out_type=jax.ShapeDtypeStruct((NNZ * BH, WIN), jnp.float32),
    mesh=_mesh,
    scratch_types=[
        pltpu.VMEM((128,), jnp.int32),
        pltpu.VMEM((128,), jnp.int32),
        pltpu.VMEM((2, RB * BH, WIN), jnp.float32),
        pltpu.SemaphoreType.DMA,
        pltpu.SemaphoreType.DMA,
        pltpu.SemaphoreType.DMA,
    ],
)
def _gather_windows(x, r2d, c2d, out, rows_v, cols_v, wide,
                    sem_in0, sem_in1, sem_out):
    wid = lax.axis_index("s") * 2 + lax.axis_index("c")
    pltpu.sync_copy(r2d.at[wid], rows_v)
    pltpu.sync_copy(c2d.at[wid], cols_v)

    rc = []
    for j in range((PER_W + 15) // 16):
        r16 = rows_v[pl.ds(j * 16, 16)]
        c16 = cols_v[pl.ds(j * 16, 16)]
        for k in range(16):
            if j * 16 + k >= PER_W:
                break
            rc.append((r16[k], c16[k]))

    def in_copy(g, m):
        r, c = rc[g * RB + m]
        return pltpu.make_async_copy(
            x.at[pl.ds(r * BH, BH), pl.ds((c >> 2) * WIN, WIN)],
            wide.at[g % 2, pl.ds(m * BH, BH)], sem_in1 if g % 2 else sem_in0)

    def out_copy(g):
        row0 = (wid * PER_W + g * RB) * BH
        return pltpu.make_async_copy(
            wide.at[g % 2], out.at[pl.ds(row0, RB * BH)], sem_out)

    for m in range(RB):
        in_copy(0, m).start()
    for g in range(NROUNDS):
        if g + 1 < NROUNDS:
            for m in range(RB):
                in_copy(g + 1, m).start()
        for m in range(RB):
            in_copy(g, m).wait()
        if g < 2:
            out_copy(g).start()
            out_copy(g).wait()
        else:
            @pl.when(wid < NW - 1)
            def _():
                out_copy(g).start()
                out_copy(g).wait()


_TC_GRID = 39
_TB = NNZ // _TC_GRID


def _extract_body(o_ref, in_ref, out_ref):
    x = in_ref[...]
    o = o_ref[...]
    y = x[:, 0:BW]
    for q in (1, 2, 3):
        y = jnp.where(o == q, x[:, q * BW:(q + 1) * BW], y)
    out_ref[...] = y.reshape(_TB, BH, BW)


_extract_call = pl.pallas_call(
    _extract_body,
    grid=(_TC_GRID,),
    in_specs=[
        pl.BlockSpec((_TB * BH, 1), lambda i: (i, 0)),
        pl.BlockSpec((_TB * BH, WIN), lambda i: (i, 0)),
    ],
    out_specs=pl.BlockSpec((_TB, BH, BW), lambda i: (i, 0, 0)),
    out_shape=jax.ShapeDtypeStruct((NNZ, BH, BW), jnp.float32),
)


def kernel(x, active_indices):
    ai = active_indices.astype(jnp.int32)
    pad = jnp.zeros((NW * PER_W, 2), jnp.int32).at[:NNZ].set(ai)
    r2d = jnp.zeros((NW, 128), jnp.int32).at[:, :PER_W].set(
        pad[:, 0].reshape(NW, PER_W))
    c2d = jnp.zeros((NW, 128), jnp.int32).at[:, :PER_W].set(
        pad[:, 1].reshape(NW, PER_W))
    wide = _gather_windows(x, r2d, c2d)
    o_rows = jnp.repeat(ai[:, 1] % 4, BH)[:, None]
    return _extract_call(o_rows, wide)


# speedup vs baseline: 3.7244x; 1.8926x over previous
import functools

import jax
import jax.numpy as jnp
from jax import lax
from jax.experimental import pallas as pl
from jax.experimental.pallas import tpu as pltpu, tpu_sc as plsc

N = 4096
BH = BW = 32
GRID = N // BH
NNZ = 1638
NW = 32
PER_W = 52
RB = 13
NROUNDS = PER_W // RB
BLK = BH * BW

_mesh = plsc.VectorSubcoreMesh(core_axis_name="c", subcore_axis_name="s")


@functools.partial(
    pl.kernel,
    out_type=jax.ShapeDtypeStruct((NNZ * BLK,), jnp.float32),
    mesh=_mesh,
    scratch_types=[
        pltpu.VMEM((128,), jnp.int32),
        pltpu.VMEM((128,), jnp.int32),
        pltpu.VMEM((2 * RB, BH, 4 * BW), jnp.float32),
        pltpu.VMEM((RB * BLK,), jnp.float32),
        pltpu.SemaphoreType.DMA,
        pltpu.SemaphoreType.DMA,
        pltpu.SemaphoreType.DMA,
    ],
)
def _gather_blocks(x, r2d, c2d, out, rows_v, cols_v, wide, stage,
                   sem_in0, sem_in1, sem_out):
    wid = lax.axis_index("s") * 2 + lax.axis_index("c")
    pltpu.sync_copy(r2d.at[wid], rows_v)
    pltpu.sync_copy(c2d.at[wid], cols_v)

    rc = []
    for j in range((PER_W + 15) // 16):
        r16 = rows_v[pl.ds(j * 16, 16)]
        c16 = cols_v[pl.ds(j * 16, 16)]
        for k in range(16):
            if j * 16 + k >= PER_W:
                break
            rc.append((r16[k], c16[k]))

    def in_copy(g, m):
        r, c = rc[g * RB + m]
        return pltpu.make_async_copy(
            x.at[pl.ds(r * BH, BH), pl.ds((c >> 2) * (4 * BW), 4 * BW)],
            wide.at[(g % 2) * RB + m], sem_in1 if g % 2 else sem_in0)

    def out_copy(g):
        return pltpu.make_async_copy(
            stage, out.at[pl.ds((wid * PER_W + g * RB) * BLK, RB * BLK)],
            sem_out)

    for m in range(RB):
        in_copy(0, m).start()
    for g in range(NROUNDS):
        if g + 1 < NROUNDS:
            for m in range(RB):
                in_copy(g + 1, m).start()
        for m in range(RB):
            in_copy(g, m).wait()

        offs = [(c & 3) * BW for _, c in rc[g * RB:(g + 1) * RB]]
        slot0 = (g % 2) * RB

        def extract_row(i, _):
            for m in range(RB):
                src = wide.at[slot0 + m]
                dst_base = m * BLK + i * BW
                for h in (0, 16):
                    stage[pl.ds(dst_base + h, 16)] = src[i, pl.ds(offs[m] + h, 16)]
            return _

        valid = (wid < NW - 1) if g >= 2 else None
        if valid is None:
            lax.fori_loop(0, BH, extract_row, 0, unroll=4)
            out_copy(g).start()
            out_copy(g).wait()
        else:
            @pl.when(valid)
            def _():
                lax.fori_loop(0, BH, extract_row, 0, unroll=4)
                out_copy(g).start()
                out_copy(g).wait()


def kernel(x, active_indices):
    ai = active_indices.astype(jnp.int32)
    pad = jnp.zeros((NW * PER_W, 2), jnp.int32).at[:NNZ].set(ai)
    r2d = jnp.zeros((NW, 128), jnp.int32).at[:, :PER_W].set(
        pad[:, 0].reshape(NW, PER_W))
    c2d = jnp.zeros((NW, 128), jnp.int32).at[:, :PER_W].set(
        pad[:, 1].reshape(NW, PER_W))
    flat = _gather_blocks(x, r2d, c2d)
    return flat.reshape(NNZ, BH, BW)
